# baseline (speedup 1.0000x reference)
import jax
import jax.numpy as jnp
from jax import lax
from jax.experimental import pallas as pl
from jax.experimental.pallas import tpu as pltpu
from jax.experimental.pallas import tpu_sc as plsc

_N = 10000
_E = 320000
_D = 128
_NC = 2
_NS = 16
_NW = _NC * _NS
_CHUNK = 80
_NCHUNK = 126
_EP = _NW * _NCHUNK * _CHUNK
_NP = 10240
_RPT = _NP // _NS


def _ilv_bf16(x):
    r = x.reshape(x.shape[0], _D // 32, 2, 16)
    r = jnp.swapaxes(r, 2, 3)
    return r.reshape(x.shape[0], _D).astype(jnp.bfloat16)


def _mlp_body(x_ref, w1, b1, w2, b2, w3, b3, w4, b4, x1b_ref, m2_ref):
    xb = x_ref[...]
    dn = (((1,), (1,)), ((), ()))
    h = jnp.maximum(
        lax.dot_general(xb, w1[...], dn, preferred_element_type=jnp.float32)
        + b1[...], 0.0)
    x1 = jnp.maximum(
        lax.dot_general(h, w2[...], dn, preferred_element_type=jnp.float32)
        + b2[...], 0.0)
    x1b_ref[...] = _ilv_bf16(x1)
    g = jnp.maximum(
        lax.dot_general(xb, w3[...], dn, preferred_element_type=jnp.float32)
        + b3[...], 0.0)
    m2_ref[...] = jnp.maximum(
        lax.dot_general(g, w4[...], dn, preferred_element_type=jnp.float32)
        + b4[...], 0.0)


def _mlps(x2d, W1, b1, W2, b2, W3, b3, W4, b4):
    BM = 2000
    wspec = pl.BlockSpec((_D, _D), lambda i: (0, 0))
    bspec = pl.BlockSpec((1, _D), lambda i: (0, 0))
    rspec = pl.BlockSpec((BM, _D), lambda i: (i, 0))
    return pl.pallas_call(
        _mlp_body,
        grid=(_N // BM,),
        in_specs=[rspec, wspec, bspec, wspec, bspec, wspec, bspec, wspec, bspec],
        out_specs=[rspec, rspec],
        out_shape=[jax.ShapeDtypeStruct((_N, _D), jnp.bfloat16),
                   jax.ShapeDtypeStruct((_N, _D), jnp.float32)],
    )(x2d, W1, b1.reshape(1, _D), W2, b2.reshape(1, _D),
      W3, b3.reshape(1, _D), W4, b4.reshape(1, _D))


def _add2_body(a_ref, b_ref, ob_ref):
    ob_ref[...] = _ilv_bf16(a_ref[...] + b_ref[...])


def _add3_body(a_ref, b_ref, c_ref, o_ref):
    o_ref[...] = a_ref[...] + b_ref[...] + c_ref[...]


def _combine2(a, b):
    BM = 2000
    rspec = pl.BlockSpec((BM, _D), lambda i: (i, 0))
    return pl.pallas_call(
        _add2_body,
        grid=(_N // BM,),
        in_specs=[rspec, rspec],
        out_specs=rspec,
        out_shape=jax.ShapeDtypeStruct((_N, _D), jnp.bfloat16),
    )(a, b)


def _combine3(a, b, c):
    BM = 2000
    rspec = pl.BlockSpec((BM, _D), lambda i: (i, 0))
    return pl.pallas_call(
        _add3_body,
        grid=(_N // BM,),
        in_specs=[rspec, rspec, rspec],
        out_specs=rspec,
        out_shape=jax.ShapeDtypeStruct((_N, _D), jnp.float32),
    )(a, b, c)


def _spmm_body(meta_hbm, x_hbm, out_hbm,
               meta, srci, dsti, rows, scl, acc_sh, gsem, ssem, msem):
    c = lax.axis_index("c")
    s = lax.axis_index("s")
    w = c * _NS + s

    zvec = jnp.zeros((16,), jnp.float32)

    def zbody(i, carry):
        scl[0][i // 8, pl.ds((i % 8) * 16, 16)] = zvec
        return carry

    lax.fori_loop(0, _CHUNK * 8, zbody, 0)
    r0 = s * _RPT
    for t in range(_RPT // _CHUNK):
        pltpu.sync_copy(scl[0], acc_sh.at[pl.ds(r0 + t * _CHUNK, _CHUNK)])
    plsc.subcore_barrier()

    def meta_load(j, m3):
        pltpu.async_copy(meta_hbm.at[w].at[j], meta[m3], msem[m3])

    def wait_meta(j, m3):
        pltpu.make_async_copy(meta_hbm.at[w].at[j], meta[m3], msem[m3]).wait()

    def cvt_idx(m3):
        for g in range(_CHUNK // 16):
            sl = pl.ds(g * 16, 16)
            srci[m3][0, sl] = meta[m3][0, sl].astype(jnp.int32)
            dsti[m3][0, sl] = meta[m3][1, sl].astype(jnp.int32)

    def gather(j, b2, m3):
        pltpu.async_copy(x_hbm.at[srci[m3].at[0]], rows[b2], gsem[b2])

    def wait_gather(j, b2, m3):
        pltpu.make_async_copy(
            x_hbm.at[srci[m3].at[0]], rows[b2], gsem[b2]).wait()

    def scatter(b2, m3):
        pltpu.async_copy(scl[b2], acc_sh.at[dsti[m3].at[0]],
                         ssem[b2], add=True)

    def wait_scatter(b2, m3):
        pltpu.make_async_copy(
            scl[b2], acc_sh.at[dsti[m3].at[0]], ssem[b2]).wait()

    def scale(b2, m3):
        r = rows[b2]
        o = scl[b2]

        def group(g, carry):
            vv = meta[m3][2, pl.ds(g * 16, 16)]
            for l in range(16):
                v = vv[l]
                e = g * 16 + l
                for h in range(_D // 32):
                    q = r[e, pl.ds(h * 16, 16)]
                    ab = plsc.bitcast(q, jnp.bfloat16)
                    a, b = plsc.unpack(ab, format=plsc.PackFormat.INTERLEAVED)
                    o[e, pl.ds(h * 32, 16)] = a * v
                    o[e, pl.ds(h * 32 + 16, 16)] = b * v
            return carry

        lax.fori_loop(0, _CHUNK // 16, group, 0)

    meta_load(0, 0)
    wait_meta(0, 0)
    cvt_idx(0)
    gather(0, 0, 0)
    meta_load(1, 1)
    nq = _NCHUNK // 6

    def chunk_step(j, q, t):
        b2 = q % 2
        m3 = q % 3
        nb2 = (q + 1) % 2
        nm3 = (q + 1) % 3

        if q >= 2:
            wait_scatter(b2, nm3)
        else:
            @pl.when(t > 0)
            def _():
                wait_scatter(b2, nm3)

        def launch(load_next):
            wait_meta(j + 1, nm3)
            cvt_idx(nm3)
            gather(j + 1, nb2, nm3)
            if load_next:
                meta_load(j + 2, (q + 2) % 3)

        if q < 4:
            launch(True)
        elif q == 4:
            launch(False)

            @pl.when(t < nq - 1)
            def _():
                meta_load(j + 2, (q + 2) % 3)
        else:
            @pl.when(t < nq - 1)
            def _():
                launch(True)

        wait_gather(j, b2, m3)
        scale(b2, m3)
        scatter(b2, m3)

    def body(t, carry):
        for q in range(6):
            chunk_step(6 * t + q, q, t)
        return carry

    lax.fori_loop(0, nq, body, 0)
    wait_scatter(0, 1)
    wait_scatter(1, 2)
    plsc.subcore_barrier()
    pltpu.sync_copy(acc_sh.at[pl.ds(r0, _RPT)],
                    out_hbm.at[pl.ds(c * _NP + r0, _RPT)])


def _spmm_partials(meta, x_bf):
    mesh = plsc.VectorSubcoreMesh(
        core_axis_name="c", subcore_axis_name="s",
        num_cores=_NC, num_subcores=_NS)
    kern = pl.kernel(
        _spmm_body,
        out_type=jax.ShapeDtypeStruct((_NC * _NP, _D), jnp.float32),
        mesh=mesh,
        compiler_params=pltpu.CompilerParams(
            needs_layout_passes=False, use_tc_tiling_on_sc=False),
        scratch_types=[
            [pltpu.VMEM((3, _CHUNK), jnp.float32)] * 3,
            [pltpu.VMEM((1, _CHUNK), jnp.int32)] * 3,
            [pltpu.VMEM((1, _CHUNK), jnp.int32)] * 3,
            [pltpu.VMEM((_CHUNK, _D // 2), jnp.int32)] * 2,
            [pltpu.VMEM((_CHUNK, _D), jnp.float32)] * 2,
            pltpu.VMEM_SHARED((_NP, _D), jnp.float32),
            [pltpu.SemaphoreType.DMA] * 2,
            [pltpu.SemaphoreType.DMA] * 2,
            [pltpu.SemaphoreType.DMA] * 3,
        ],
    )
    return kern(meta, x_bf)


def _edge_meta(idx, val):
    npad = _EP - _E
    pad_i = (jnp.arange(npad, dtype=jnp.int32) % _N).astype(jnp.float32)
    srcf = jnp.concatenate([idx[1].astype(jnp.float32), pad_i])
    dstf = jnp.concatenate([idx[0].astype(jnp.float32), pad_i])
    vf = jnp.concatenate([val, jnp.zeros((npad,), jnp.float32)])
    sh = (_NW, _NCHUNK, 1, _CHUNK)
    return jnp.concatenate(
        [srcf.reshape(sh), dstf.reshape(sh), vf.reshape(sh)], axis=2)


def kernel(K_value, index, normed_A_value, A_index, x, n1, n2,
           W1, b1, W2, b2, W3, b3, W4, b4):
    x2d = x.reshape(_N, _D)
    x1b, m2 = _mlps(x2d, W1, b1, W2, b2, W3, b3, W4, b4)
    x1v = lax.bitcast_convert_type(
        x1b.reshape(_N, _D // 2, 2), jnp.int32)

    p = _spmm_partials(_edge_meta(index, K_value), x1v)
    wxb = _combine2(p[:_N], p[_NP:_NP + _N])
    wxv = lax.bitcast_convert_type(
        wxb.reshape(_N, _D // 2, 2), jnp.int32)

    q = _spmm_partials(_edge_meta(A_index, normed_A_value), wxv)
    out = _combine3(q[:_N], q[_NP:_NP + _N], m2)
    return out[None]


# speedup vs baseline: 2.3813x; 2.3813x over previous
import jax
import jax.numpy as jnp
from jax import lax
from jax.experimental import pallas as pl
from jax.experimental.pallas import tpu as pltpu
from jax.experimental.pallas import tpu_sc as plsc

_N = 10000
_E = 320000
_D = 128
_NC = 2
_NS = 16
_NW = _NC * _NS
_CHUNK = 96
_NCHUNK = 107
_EP = _NW * _NCHUNK * _CHUNK
_NP = 10240
_RPT = _NP // _NS


def _mlp1_body(x_ref, w1, b1, w2, b2, x1_ref):
    xb = x_ref[...]
    dn = (((1,), (1,)), ((), ()))
    h = jnp.maximum(
        lax.dot_general(xb, w1[...], dn, preferred_element_type=jnp.float32)
        + b1[...], 0.0)
    x1_ref[...] = jnp.maximum(
        lax.dot_general(h, w2[...], dn, preferred_element_type=jnp.float32)
        + b2[...], 0.0)


_BM = 2000
_wspec = pl.BlockSpec((_D, _D), lambda i: (0, 0))
_bspec = pl.BlockSpec((1, _D), lambda i: (0, 0))
_rspec = pl.BlockSpec((_BM, _D), lambda i: (i, 0))


def _mlp1(x2d, W1, b1, W2, b2):
    return pl.pallas_call(
        _mlp1_body,
        grid=(_N // _BM,),
        in_specs=[_rspec, _wspec, _bspec, _wspec, _bspec],
        out_specs=_rspec,
        out_shape=jax.ShapeDtypeStruct((_N, _D), jnp.float32),
    )(x2d, W1, b1.reshape(1, _D), W2, b2.reshape(1, _D))


def _add2_body(a_ref, b_ref, o_ref):
    o_ref[...] = a_ref[...] + b_ref[...]


def _combine(parts):
    return pl.pallas_call(
        _add2_body,
        grid=(_N // _BM,),
        in_specs=[_rspec] * 2,
        out_specs=_rspec,
        out_shape=jax.ShapeDtypeStruct((_N, _D), jnp.float32),
    )(parts[0], parts[1])


def _final_body(a_ref, b_ref, x_ref, w3, b3, w4, b4, o_ref):
    xb = x_ref[...]
    dn = (((1,), (1,)), ((), ()))
    g = jnp.maximum(
        lax.dot_general(xb, w3[...], dn, preferred_element_type=jnp.float32)
        + b3[...], 0.0)
    m2 = jnp.maximum(
        lax.dot_general(g, w4[...], dn, preferred_element_type=jnp.float32)
        + b4[...], 0.0)
    o_ref[...] = a_ref[...] + b_ref[...] + m2


def _final(a, b, x2d, W3, b3, W4, b4):
    return pl.pallas_call(
        _final_body,
        grid=(_N // _BM,),
        in_specs=[_rspec, _rspec, _rspec, _wspec, _bspec, _wspec, _bspec],
        out_specs=_rspec,
        out_shape=jax.ShapeDtypeStruct((_N, _D), jnp.float32),
    )(a, b, x2d, W3, b3.reshape(1, _D), W4, b4.reshape(1, _D))


def _spmm_body(meta_hbm, x_hbm, out_hbm,
               meta, srci, dsti, rows, acc_sh, gsem, ssem, msem):
    c = lax.axis_index("c")
    s = lax.axis_index("s")
    w = c * _NS + s

    zvec = jnp.zeros((16,), jnp.float32)

    def zbody(i, carry):
        rows[0][i // 8, pl.ds((i % 8) * 16, 16)] = zvec
        return carry

    lax.fori_loop(0, _CHUNK * 8, zbody, 0)
    r0 = s * _RPT
    for t in range(_RPT // _CHUNK):
        pltpu.sync_copy(rows[0], acc_sh.at[pl.ds(r0 + t * _CHUNK, _CHUNK)])
    _rem = _RPT - (_RPT // _CHUNK) * _CHUNK
    if _rem:
        pltpu.sync_copy(
            rows[0].at[pl.ds(0, _rem)],
            acc_sh.at[pl.ds(r0 + (_RPT // _CHUNK) * _CHUNK, _rem)])
    plsc.subcore_barrier()

    def meta_load(j, b):
        pltpu.async_copy(meta_hbm.at[w].at[j], meta[b], msem[b])

    def wait_meta(j, b):
        pltpu.make_async_copy(meta_hbm.at[w].at[j], meta[b], msem[b]).wait()

    def cvt_idx(b):
        for g in range(_CHUNK // 16):
            sl = pl.ds(g * 16, 16)
            srci[b][0, sl] = meta[b][0, sl].astype(jnp.int32)
            dsti[b][0, sl] = meta[b][1, sl].astype(jnp.int32)

    def gather(j, b):
        pltpu.async_copy(x_hbm.at[srci[b].at[0]], rows[b], gsem[b])

    def wait_gather(j, b):
        pltpu.make_async_copy(
            x_hbm.at[srci[b].at[0]], rows[b], gsem[b]).wait()

    def scatter(b):
        pltpu.async_copy(rows[b], acc_sh.at[dsti[b].at[0]], ssem[b], add=True)

    def wait_scatter(b):
        pltpu.make_async_copy(
            rows[b], acc_sh.at[dsti[b].at[0]], ssem[b]).wait()

    def scale(b):
        r = rows[b]
        for g in range(_CHUNK // 16):
            vv = meta[b][2, pl.ds(g * 16, 16)]
            for l in range(16):
                e = g * 16 + l
                v = vv[l]
                for k in range(_D // 16):
                    r[e, pl.ds(k * 16, 16)] = r[e, pl.ds(k * 16, 16)] * v

    meta_load(0, 0)
    wait_meta(0, 0)
    cvt_idx(0)
    gather(0, 0)
    meta_load(1, 1)
    nq = (_NCHUNK - 2) // 3

    def chunk_step(j, b, launch=True, load_next=True):
        nb = (b + 1) % 3

        if isinstance(j, int):
            if j >= 2:
                wait_scatter(nb)
        else:
            @pl.when(j >= 2)
            def _():
                wait_scatter(nb)

        if launch:
            wait_meta(j + 1, nb)
            cvt_idx(nb)
            gather(j + 1, nb)
            if load_next:
                meta_load(j + 2, (nb + 1) % 3)

        wait_gather(j, b)
        scale(b)
        scatter(b)

    def body(t, carry):
        for q in range(3):
            chunk_step(3 * t + q, q)
        return carry

    lax.fori_loop(0, nq, body, 0)
    chunk_step(_NCHUNK - 2, 0, load_next=False)
    chunk_step(_NCHUNK - 1, 1, launch=False)
    wait_scatter(0)
    wait_scatter(1)
    plsc.subcore_barrier()
    pltpu.sync_copy(acc_sh.at[pl.ds(r0, _RPT)],
                    out_hbm.at[pl.ds(c * _NP + r0, _RPT)])


def _spmm_partials(meta, x_mat):
    mesh = plsc.VectorSubcoreMesh(
        core_axis_name="c", subcore_axis_name="s",
        num_cores=_NC, num_subcores=_NS)
    kern = pl.kernel(
        _spmm_body,
        out_type=jax.ShapeDtypeStruct((_NC * _NP, _D), jnp.float32),
        mesh=mesh,
        scratch_types=[
            [pltpu.VMEM((3, _CHUNK), jnp.float32)] * 3,
            [pltpu.VMEM((1, _CHUNK), jnp.int32)] * 3,
            [pltpu.VMEM((1, _CHUNK), jnp.int32)] * 3,
            [pltpu.VMEM((_CHUNK, _D), jnp.float32)] * 3,
            pltpu.VMEM_SHARED((_NP, _D), jnp.float32),
            [pltpu.SemaphoreType.DMA] * 3,
            [pltpu.SemaphoreType.DMA] * 3,
            [pltpu.SemaphoreType.DMA] * 3,
        ],
    )
    return kern(meta, x_mat)


def _edge_meta(idx, val):
    npad = _EP - _E
    sh = (_NW, _NCHUNK, 1, _CHUNK)
    pad_i = (jnp.arange(npad, dtype=jnp.int32) % _N).astype(jnp.float32)
    srcf = jnp.concatenate([idx[1].astype(jnp.float32), pad_i]).reshape(sh)
    dstf = jnp.concatenate([idx[0].astype(jnp.float32), pad_i]).reshape(sh)
    vf = jnp.concatenate([val, jnp.zeros((npad,), jnp.float32)]).reshape(sh)
    return jnp.concatenate([srcf, dstf, vf], axis=2)


def kernel(K_value, index, normed_A_value, A_index, x, n1, n2,
           W1, b1, W2, b2, W3, b3, W4, b4):
    x2d = x.reshape(_N, _D)
    x1 = _mlp1(x2d, W1, b1, W2, b2)

    p = _spmm_partials(_edge_meta(index, K_value), x1)
    wx = _combine((p[:_N], p[_NP:_NP + _N]))

    q = _spmm_partials(_edge_meta(A_index, normed_A_value), wx)
    out = _final(q[:_N], q[_NP:_NP + _N], x2d, W3, b3, W4, b4)
    return out[None]
